# baseline (speedup 1.0000x reference)
import functools

import jax
import jax.numpy as jnp
from jax import lax
from jax.experimental import pallas as pl
from jax.experimental.pallas import tpu as pltpu
from jax.experimental.pallas import tpu_sc as plsc

N = 10000
E = 320000
D = 128
NP = 10240

F32 = jnp.float32
BF16 = jnp.bfloat16
_DN = (((1,), (1,)), ((), ()))

_MESH = plsc.VectorSubcoreMesh(core_axis_name="c", subcore_axis_name="s")
NC, NS = 2, 16
NW = NC * NS


_K1_B = 2000


def _pack_pair(a, b):
    abits = lax.bitcast_convert_type(a.astype(BF16).astype(F32), jnp.uint32)
    bbits = lax.bitcast_convert_type(b.astype(BF16).astype(F32), jnp.uint32)
    lo = lax.shift_right_logical(abits, jnp.uint32(16))
    hi = jnp.bitwise_and(bbits, jnp.uint32(0xFFFF0000))
    return lax.bitcast_convert_type(jnp.bitwise_or(hi, lo), jnp.int32)


def _k1_body(x_ref, t_ref, wt_ref, bt_ref, wsg_ref, bsg_ref, wdg_ref, bdg_ref,
             wdu_ref, bdu_ref, wsu_ref, bsu_ref,
             tsrc_ref, tdst_ref, xup_ref):
    x = x_ref[...]
    tp = lax.dot_general(t_ref[...], wt_ref[...], _DN,
                         preferred_element_type=F32) + bt_ref[...]
    esrc = lax.dot_general(x, wsg_ref[...], _DN,
                           preferred_element_type=F32) + bsg_ref[...] + tp
    bh = lax.dot_general(x, wdu_ref[...], _DN,
                         preferred_element_type=F32) + bdu_ref[...]
    tsrc_ref[...] = _pack_pair(esrc, bh)
    tdst_ref[...] = lax.dot_general(x, wdg_ref[...], _DN,
                                    preferred_element_type=F32) + bdg_ref[...]
    xup_ref[...] = lax.dot_general(x, wsu_ref[...], _DN,
                                   preferred_element_type=F32) + bsu_ref[...]


def _k1(node_feats, time_feats, Wt, bt, Wsg, bsg, Wdg, bdg, Wdu, bdu, Wsu, bsu):
    nb = N // _K1_B
    row = lambda i: (i, 0)
    fix = lambda i: (0, 0)
    w_spec = pl.BlockSpec((D, D), fix)
    b_spec = pl.BlockSpec((1, D), fix)
    return pl.pallas_call(
        _k1_body,
        grid=(nb,),
        in_specs=[
            pl.BlockSpec((_K1_B, D), row), pl.BlockSpec((_K1_B, D), row),
            w_spec, b_spec, w_spec, b_spec, w_spec, b_spec,
            w_spec, b_spec, w_spec, b_spec,
        ],
        out_specs=[
            pl.BlockSpec((_K1_B, D), row),
            pl.BlockSpec((_K1_B, D), row),
            pl.BlockSpec((_K1_B, D), row),
        ],
        out_shape=[
            jax.ShapeDtypeStruct((N, D), jnp.int32),
            jax.ShapeDtypeStruct((N, D), F32),
            jax.ShapeDtypeStruct((N, D), F32),
        ],
    )(node_feats, time_feats, Wt, bt, Wsg, bsg, Wdg, bdg, Wdu, bdu, Wsu, bsu)


_K2_B = 40
_NBUF = 5


def _make_k2(ne, eoff):
    epw = ne // NW
    nchunks = epw // _K2_B

    @functools.partial(
        pl.kernel,
        out_type=(
            jax.ShapeDtypeStruct((ne, D), jnp.int32),
            jax.ShapeDtypeStruct((ne, D), F32),
        ),
        mesh=_MESH,
        scratch_types=(
            [pltpu.VMEM((epw,), jnp.int32)] * 2
            + [pltpu.VMEM((_K2_B, D), jnp.int32)] * _NBUF
            + [pltpu.VMEM((_K2_B, D), F32)] * _NBUF
            + [pltpu.SemaphoreType.DMA] * (4 * _NBUF)
        ),
    )
    def k2(src_hbm, dst_hbm, tsrc_hbm, tdst_hbm, gs_hbm, gd_hbm, *scr):
        isv, idv = scr[0], scr[1]
        rs = scr[2:2 + _NBUF]
        rd = scr[2 + _NBUF:2 + 2 * _NBUF]
        gssem = scr[2 + 2 * _NBUF:2 + 3 * _NBUF]
        gdsem = scr[2 + 3 * _NBUF:2 + 4 * _NBUF]
        wssem = scr[2 + 4 * _NBUF:2 + 5 * _NBUF]
        wdsem = scr[2 + 5 * _NBUF:2 + 6 * _NBUF]
        wid = lax.axis_index("s") * NC + lax.axis_index("c")
        ebase = wid * epw

        pltpu.sync_copy(src_hbm.at[pl.ds(eoff + ebase, epw)], isv)
        pltpu.sync_copy(dst_hbm.at[pl.ds(eoff + ebase, epw)], idv)

        def gather(c, b):
            sl = pl.ds(c * _K2_B, _K2_B)
            return (pltpu.make_async_copy(tsrc_hbm.at[isv.at[sl]], rs[b],
                                          gssem[b]),
                    pltpu.make_async_copy(tdst_hbm.at[idv.at[sl]], rd[b],
                                          gdsem[b]))

        def write(c, b):
            off = ebase + c * _K2_B
            return (pltpu.make_async_copy(rs[b], gs_hbm.at[pl.ds(off, _K2_B)],
                                          wssem[b]),
                    pltpu.make_async_copy(rd[b], gd_hbm.at[pl.ds(off, _K2_B)],
                                          wdsem[b]))

        for b in range(3):
            for cp in gather(b, b):
                cp.start()

        @pl.loop(0, nchunks // _NBUF)
        def _(t):
            for j in range(_NBUF):
                c = t * _NBUF + j
                bn = (j + 3) % _NBUF
                for cp in gather(c, j):
                    cp.wait()
                for cp in write(c, j):
                    cp.start()

                @pl.when(c >= 2)
                def _():
                    for cp in write(c - 2, bn):
                        cp.wait()

                @pl.when(c <= nchunks - 4)
                def _():
                    for cp in gather(c + 3, bn):
                        cp.start()

        for c in (nchunks - 2, nchunks - 1):
            for cp in write(c, c % _NBUF):
                cp.wait()

    return k2


_K3_B = 2000


def _k3_body(gs_ref, gd_ref, ef_ref, weg_ref, beg_ref, ge_ref,
             be_ref, sn_ref, y_ref):
    w = lax.bitcast_convert_type(gs_ref[...], jnp.uint32)
    esrc = lax.bitcast_convert_type(lax.shift_left(w, jnp.uint32(16)), F32)
    bh = lax.bitcast_convert_type(
        jnp.bitwise_and(w, jnp.uint32(0xFFFF0000)), F32)
    ef = ef_ref[...]
    eg = lax.dot_general(ef, weg_ref[...], _DN,
                         preferred_element_type=F32) + beg_ref[...]
    m = esrc + gd_ref[...] + eg
    sig = pl.reciprocal(1.0 + jnp.exp(-m), approx=True)
    sn_ref[0] = bh * sig
    sn_ref[1] = sig
    mu = jnp.mean(m, axis=1, keepdims=True)
    var = jnp.mean((m - mu) ** 2, axis=1, keepdims=True)
    ln = (m - mu) * lax.rsqrt(var + 1e-5) * ge_ref[...] + be_ref[...]
    y_ref[...] = ef + ln * pl.reciprocal(1.0 + jnp.exp(-ln), approx=True)


def _k3_body_alias(gs_ref, gd_ref, ef_ref, weg_ref, beg_ref, ge_ref,
                   be_ref, yin_ref, sn_ref, y_ref):
    _k3_body(gs_ref, gd_ref, ef_ref, weg_ref, beg_ref, ge_ref,
             be_ref, sn_ref, y_ref)


def _k3(gs, gd, edge_feats, Weg, beg, ge, be, ybuf, blk0):
    ne = gs.shape[0]
    nb = ne // _K3_B
    row = lambda i: (i + blk0, 0)
    crow = lambda i: (i, 0)
    fix = lambda i: (0, 0)
    in_specs = [
        pl.BlockSpec((_K3_B, D), crow),
        pl.BlockSpec((_K3_B, D), crow),
        pl.BlockSpec((_K3_B, D), row),
        pl.BlockSpec((D, D), fix),
        pl.BlockSpec((1, D), fix),
        pl.BlockSpec((1, D), fix),
        pl.BlockSpec((1, D), fix),
    ]
    args = [gs, gd, edge_feats, Weg, beg, ge, be]
    body = _k3_body
    aliases = {}
    if ybuf is not None:
        in_specs.append(pl.BlockSpec(memory_space=pl.ANY))
        args.append(ybuf)
        body = _k3_body_alias
        aliases = {7: 1}
    return pl.pallas_call(
        body,
        grid=(nb,),
        in_specs=in_specs,
        out_specs=[
            pl.BlockSpec((2, _K3_B, D), lambda i: (0, i, 0)),
            pl.BlockSpec((_K3_B, D), row),
        ],
        out_shape=[
            jax.ShapeDtypeStruct((2, ne, D), F32),
            jax.ShapeDtypeStruct((E, D), F32),
        ],
        input_output_aliases=aliases,
    )(*args)


_K4_B = 40
_ZROWS = NP // NS


def _make_k4(ne, eoff):
    eps = ne // NS
    nchunks = eps // _K4_B

    @functools.partial(
        pl.kernel,
        out_type=jax.ShapeDtypeStruct((NC, NP, D), F32),
        mesh=_MESH,
        scratch_types=(
            [pltpu.VMEM((_K4_B,), jnp.int32)] * _NBUF
            + [pltpu.VMEM((_K4_B, D), F32)] * _NBUF
            + [pltpu.VMEM_SHARED((NP, D), F32)]
            + [pltpu.SemaphoreType.DMA] * (3 * _NBUF)
        ),
    )
    def k4(dst_hbm, sn_hbm, out_hbm, *scr):
        idxv = scr[0:_NBUF]
        vals = scr[_NBUF:2 * _NBUF]
        acc = scr[2 * _NBUF]
        lsem = scr[1 + 2 * _NBUF:1 + 3 * _NBUF]
        isem = scr[1 + 3 * _NBUF:1 + 4 * _NBUF]
        ssem = scr[1 + 4 * _NBUF:1 + 5 * _NBUF]
        c = lax.axis_index("c")
        s = lax.axis_index("s")

        @pl.loop(0, _K4_B)
        def _(r):
            for j in range(D // 16):
                vals[0].at[r, pl.ds(j * 16, 16)][...] = jnp.zeros((16,), F32)

        zbase = s * _ZROWS
        @pl.loop(0, _ZROWS // _K4_B)
        def _(k):
            pltpu.sync_copy(vals[0], acc.at[pl.ds(zbase + k * _K4_B, _K4_B)])

        plsc.subcore_barrier()

        ebase = s * eps

        def load(k, b):
            off = ebase + k * _K4_B
            return (pltpu.make_async_copy(dst_hbm.at[pl.ds(eoff + off, _K4_B)],
                                          idxv[b], isem[b]),
                    pltpu.make_async_copy(sn_hbm.at[c, pl.ds(off, _K4_B)],
                                          vals[b], lsem[b]))

        def scat(k, b):
            return pltpu.make_async_copy(vals[b], acc.at[idxv[b]], ssem[b])

        for b in range(3):
            for cp in load(b, b):
                cp.start()

        @pl.loop(0, nchunks // _NBUF)
        def _(t):
            for j in range(_NBUF):
                k = t * _NBUF + j
                bn = (j + 3) % _NBUF
                for cp in load(k, j):
                    cp.wait()
                scat(k, j).start(add=True)

                @pl.when(k >= 2)
                def _():
                    scat(k - 2, bn).wait()

                @pl.when(k <= nchunks - 4)
                def _():
                    for cp in load(k + 3, bn):
                        cp.start()

        for k in (nchunks - 2, nchunks - 1):
            scat(k, k % _NBUF).wait()

        plsc.subcore_barrier()
        pltpu.sync_copy(acc.at[pl.ds(zbase, _ZROWS)],
                        out_hbm.at[c, pl.ds(zbase, _ZROWS)])

    return k4


_K5_B = 1000


def _k5_body(*refs):
    acc_refs = refs[:_NCH]
    xup_ref, nf_ref, gn_ref, bn_ref, x_ref = refs[_NCH:]
    num = acc_refs[0][0]
    den = acc_refs[0][1]
    for a in acc_refs[1:]:
        num = num + a[0]
        den = den + a[1]
    h = num / (den + 1e-6)
    xx = xup_ref[...] + h
    mu = jnp.mean(xx, axis=1, keepdims=True)
    var = jnp.mean((xx - mu) ** 2, axis=1, keepdims=True)
    ln = (xx - mu) * lax.rsqrt(var + 1e-5) * gn_ref[...] + bn_ref[...]
    x_ref[...] = nf_ref[...] + ln * jax.nn.sigmoid(ln)


def _k5(accs, xup, node_feats, gn, bn):
    nb = N // _K5_B
    row = lambda i: (i, 0)
    fix = lambda i: (0, 0)
    acc_spec = pl.BlockSpec((2, _K5_B, D), lambda i: (0, i, 0))
    return pl.pallas_call(
        _k5_body,
        grid=(nb,),
        in_specs=(
            [acc_spec] * _NCH
            + [
                pl.BlockSpec((_K5_B, D), row),
                pl.BlockSpec((_K5_B, D), row),
                pl.BlockSpec((1, D), fix),
                pl.BlockSpec((1, D), fix),
            ]
        ),
        out_specs=pl.BlockSpec((_K5_B, D), row),
        out_shape=jax.ShapeDtypeStruct((N, D), F32),
    )(*accs, xup, node_feats, gn, bn)


_CHUNKS = (32000, 96000, 96000, 64000, 32000)
_COFFS = tuple(sum(_CHUNKS[:i]) for i in range(len(_CHUNKS)))
_NCH = len(_CHUNKS)
_K2CS = tuple(_make_k2(n, o) for n, o in zip(_CHUNKS, _COFFS))
_K4CS = tuple(_make_k4(n, o) for n, o in zip(_CHUNKS, _COFFS))


def kernel(node_feats, edge_feats, time_feats, edge_index,
           Wt, bt, Wsg, bsg, Wdg, bdg, Weg, beg,
           Wsu, bsu, Wdu, bdu, gn, bn, ge, be):
    row2 = lambda v: v.reshape(1, D)
    src = edge_index[0].astype(jnp.int32)
    dst = edge_index[1].astype(jnp.int32)

    tsrc, tdst, xup = _k1(node_feats, time_feats, Wt, row2(bt), Wsg, row2(bsg),
                          Wdg, row2(bdg), Wdu, row2(bdu), Wsu, row2(bsu))

    ybuf = None
    accs = []
    for ci in range(_NCH):
        gs, gd = _K2CS[ci](src, dst, tsrc, tdst)
        sn, ybuf = _k3(gs, gd, edge_feats, Weg,
                       row2(beg), row2(ge), row2(be), ybuf,
                       _COFFS[ci] // _K3_B)
        accs.append(_K4CS[ci](dst, sn))

    x = _k5(accs, xup, node_feats, row2(gn), row2(bn))
    return (x, ybuf)


# speedup vs baseline: 1.0532x; 1.0532x over previous
import functools

import jax
import jax.numpy as jnp
from jax import lax
from jax.experimental import pallas as pl
from jax.experimental.pallas import tpu as pltpu
from jax.experimental.pallas import tpu_sc as plsc

N = 10000
E = 320000
D = 128
NP = 10240

F32 = jnp.float32
BF16 = jnp.bfloat16
_DN = (((1,), (1,)), ((), ()))

_MESH = plsc.VectorSubcoreMesh(core_axis_name="c", subcore_axis_name="s")
NC, NS = 2, 16
NW = NC * NS


_K1_B = 2000


def _pack_pair(a, b):
    abits = lax.bitcast_convert_type(a.astype(BF16).astype(F32), jnp.uint32)
    bbits = lax.bitcast_convert_type(b.astype(BF16).astype(F32), jnp.uint32)
    lo = lax.shift_right_logical(abits, jnp.uint32(16))
    hi = jnp.bitwise_and(bbits, jnp.uint32(0xFFFF0000))
    return lax.bitcast_convert_type(jnp.bitwise_or(hi, lo), jnp.int32)


def _k1_body(x_ref, t_ref, wt_ref, bt_ref, wsg_ref, bsg_ref, wdg_ref, bdg_ref,
             wdu_ref, bdu_ref, wsu_ref, bsu_ref,
             tsrc_ref, tdst_ref, xup_ref):
    x = x_ref[...]
    tp = lax.dot_general(t_ref[...], wt_ref[...], _DN,
                         preferred_element_type=F32) + bt_ref[...]
    esrc = lax.dot_general(x, wsg_ref[...], _DN,
                           preferred_element_type=F32) + bsg_ref[...] + tp
    bh = lax.dot_general(x, wdu_ref[...], _DN,
                         preferred_element_type=F32) + bdu_ref[...]
    tsrc_ref[...] = _pack_pair(esrc, bh)
    tdst_ref[...] = lax.dot_general(x, wdg_ref[...], _DN,
                                    preferred_element_type=F32) + bdg_ref[...]
    xup_ref[...] = lax.dot_general(x, wsu_ref[...], _DN,
                                   preferred_element_type=F32) + bsu_ref[...]


def _k1(node_feats, time_feats, Wt, bt, Wsg, bsg, Wdg, bdg, Wdu, bdu, Wsu, bsu):
    nb = N // _K1_B
    row = lambda i: (i, 0)
    fix = lambda i: (0, 0)
    w_spec = pl.BlockSpec((D, D), fix)
    b_spec = pl.BlockSpec((1, D), fix)
    return pl.pallas_call(
        _k1_body,
        grid=(nb,),
        in_specs=[
            pl.BlockSpec((_K1_B, D), row), pl.BlockSpec((_K1_B, D), row),
            w_spec, b_spec, w_spec, b_spec, w_spec, b_spec,
            w_spec, b_spec, w_spec, b_spec,
        ],
        out_specs=[
            pl.BlockSpec((_K1_B, D), row),
            pl.BlockSpec((_K1_B, D), row),
            pl.BlockSpec((_K1_B, D), row),
        ],
        out_shape=[
            jax.ShapeDtypeStruct((N, D), jnp.int32),
            jax.ShapeDtypeStruct((N, D), F32),
            jax.ShapeDtypeStruct((N, D), F32),
        ],
    )(node_feats, time_feats, Wt, bt, Wsg, bsg, Wdg, bdg, Wdu, bdu, Wsu, bsu)


_K2_B = 40
_NBUF = 5


def _make_k2(ne, eoff):
    epw = ne // NW
    nchunks = epw // _K2_B

    @functools.partial(
        pl.kernel,
        out_type=(
            jax.ShapeDtypeStruct((ne, D), jnp.int32),
            jax.ShapeDtypeStruct((ne, D), F32),
        ),
        mesh=_MESH,
        scratch_types=(
            [pltpu.VMEM((epw,), jnp.int32)] * 2
            + [pltpu.VMEM((_K2_B, D), jnp.int32)] * _NBUF
            + [pltpu.VMEM((_K2_B, D), F32)] * _NBUF
            + [pltpu.SemaphoreType.DMA] * (4 * _NBUF)
        ),
    )
    def k2(src_hbm, dst_hbm, tsrc_hbm, tdst_hbm, gs_hbm, gd_hbm, *scr):
        isv, idv = scr[0], scr[1]
        rs = scr[2:2 + _NBUF]
        rd = scr[2 + _NBUF:2 + 2 * _NBUF]
        gssem = scr[2 + 2 * _NBUF:2 + 3 * _NBUF]
        gdsem = scr[2 + 3 * _NBUF:2 + 4 * _NBUF]
        wssem = scr[2 + 4 * _NBUF:2 + 5 * _NBUF]
        wdsem = scr[2 + 5 * _NBUF:2 + 6 * _NBUF]
        wid = lax.axis_index("s") * NC + lax.axis_index("c")
        ebase = wid * epw

        pltpu.sync_copy(src_hbm.at[pl.ds(eoff + ebase, epw)], isv)
        pltpu.sync_copy(dst_hbm.at[pl.ds(eoff + ebase, epw)], idv)

        def gather(c, b):
            sl = pl.ds(c * _K2_B, _K2_B)
            return (pltpu.make_async_copy(tsrc_hbm.at[isv.at[sl]], rs[b],
                                          gssem[b]),
                    pltpu.make_async_copy(tdst_hbm.at[idv.at[sl]], rd[b],
                                          gdsem[b]))

        def write(c, b):
            off = ebase + c * _K2_B
            return (pltpu.make_async_copy(rs[b], gs_hbm.at[pl.ds(off, _K2_B)],
                                          wssem[b]),
                    pltpu.make_async_copy(rd[b], gd_hbm.at[pl.ds(off, _K2_B)],
                                          wdsem[b]))

        for b in range(3):
            for cp in gather(b, b):
                cp.start()

        @pl.loop(0, nchunks // _NBUF)
        def _(t):
            for j in range(_NBUF):
                c = t * _NBUF + j
                bn = (j + 3) % _NBUF
                for cp in gather(c, j):
                    cp.wait()
                for cp in write(c, j):
                    cp.start()

                @pl.when(c >= 2)
                def _():
                    for cp in write(c - 2, bn):
                        cp.wait()

                @pl.when(c <= nchunks - 4)
                def _():
                    for cp in gather(c + 3, bn):
                        cp.start()

        for c in (nchunks - 2, nchunks - 1):
            for cp in write(c, c % _NBUF):
                cp.wait()

    return k2


_K3_B = 4000


def _k3_body(gs_ref, gd_ref, ef_ref, weg_ref, beg_ref, ge_ref,
             be_ref, sn_ref, y_ref):
    w = lax.bitcast_convert_type(gs_ref[...], jnp.uint32)
    esrc = lax.bitcast_convert_type(lax.shift_left(w, jnp.uint32(16)), F32)
    bh = lax.bitcast_convert_type(
        jnp.bitwise_and(w, jnp.uint32(0xFFFF0000)), F32)
    ef = ef_ref[...]
    eg = lax.dot_general(ef, weg_ref[...], _DN,
                         preferred_element_type=F32) + beg_ref[...]
    m = esrc + gd_ref[...] + eg
    sig = pl.reciprocal(1.0 + jnp.exp(-m), approx=True)
    sn_ref[0] = bh * sig
    sn_ref[1] = sig
    mu = jnp.mean(m, axis=1, keepdims=True)
    var = jnp.mean((m - mu) ** 2, axis=1, keepdims=True)
    ln = (m - mu) * lax.rsqrt(var + 1e-5) * ge_ref[...] + be_ref[...]
    y_ref[...] = ef + ln * pl.reciprocal(1.0 + jnp.exp(-ln), approx=True)


def _k3_body_alias(gs_ref, gd_ref, ef_ref, weg_ref, beg_ref, ge_ref,
                   be_ref, yin_ref, sn_ref, y_ref):
    _k3_body(gs_ref, gd_ref, ef_ref, weg_ref, beg_ref, ge_ref,
             be_ref, sn_ref, y_ref)


def _k3(gs, gd, edge_feats, Weg, beg, ge, be, ybuf, blk0):
    ne = gs.shape[0]
    nb = ne // _K3_B
    row = lambda i: (i + blk0, 0)
    crow = lambda i: (i, 0)
    fix = lambda i: (0, 0)
    in_specs = [
        pl.BlockSpec((_K3_B, D), crow),
        pl.BlockSpec((_K3_B, D), crow),
        pl.BlockSpec((_K3_B, D), row),
        pl.BlockSpec((D, D), fix),
        pl.BlockSpec((1, D), fix),
        pl.BlockSpec((1, D), fix),
        pl.BlockSpec((1, D), fix),
    ]
    args = [gs, gd, edge_feats, Weg, beg, ge, be]
    body = _k3_body
    aliases = {}
    if ybuf is not None:
        in_specs.append(pl.BlockSpec(memory_space=pl.ANY))
        args.append(ybuf)
        body = _k3_body_alias
        aliases = {7: 1}
    return pl.pallas_call(
        body,
        grid=(nb,),
        in_specs=in_specs,
        out_specs=[
            pl.BlockSpec((2, _K3_B, D), lambda i: (0, i, 0)),
            pl.BlockSpec((_K3_B, D), row),
        ],
        out_shape=[
            jax.ShapeDtypeStruct((2, ne, D), F32),
            jax.ShapeDtypeStruct((E, D), F32),
        ],
        input_output_aliases=aliases,
    )(*args)


_K4_B = 40
_ZROWS = NP // NS


def _make_k4(ne, eoff):
    eps = ne // NS
    nchunks = eps // _K4_B

    @functools.partial(
        pl.kernel,
        out_type=jax.ShapeDtypeStruct((NC, NP, D), F32),
        mesh=_MESH,
        scratch_types=(
            [pltpu.VMEM((_K4_B,), jnp.int32)] * _NBUF
            + [pltpu.VMEM((_K4_B, D), F32)] * _NBUF
            + [pltpu.VMEM_SHARED((NP, D), F32)]
            + [pltpu.SemaphoreType.DMA] * (3 * _NBUF)
        ),
    )
    def k4(dst_hbm, sn_hbm, out_hbm, *scr):
        idxv = scr[0:_NBUF]
        vals = scr[_NBUF:2 * _NBUF]
        acc = scr[2 * _NBUF]
        lsem = scr[1 + 2 * _NBUF:1 + 3 * _NBUF]
        isem = scr[1 + 3 * _NBUF:1 + 4 * _NBUF]
        ssem = scr[1 + 4 * _NBUF:1 + 5 * _NBUF]
        c = lax.axis_index("c")
        s = lax.axis_index("s")

        @pl.loop(0, _K4_B)
        def _(r):
            for j in range(D // 16):
                vals[0].at[r, pl.ds(j * 16, 16)][...] = jnp.zeros((16,), F32)

        zbase = s * _ZROWS
        @pl.loop(0, _ZROWS // _K4_B)
        def _(k):
            pltpu.sync_copy(vals[0], acc.at[pl.ds(zbase + k * _K4_B, _K4_B)])

        plsc.subcore_barrier()

        ebase = s * eps

        def load(k, b):
            off = ebase + k * _K4_B
            return (pltpu.make_async_copy(dst_hbm.at[pl.ds(eoff + off, _K4_B)],
                                          idxv[b], isem[b]),
                    pltpu.make_async_copy(sn_hbm.at[c, pl.ds(off, _K4_B)],
                                          vals[b], lsem[b]))

        def scat(k, b):
            return pltpu.make_async_copy(vals[b], acc.at[idxv[b]], ssem[b])

        for b in range(3):
            for cp in load(b, b):
                cp.start()

        @pl.loop(0, nchunks // _NBUF)
        def _(t):
            for j in range(_NBUF):
                k = t * _NBUF + j
                bn = (j + 3) % _NBUF
                for cp in load(k, j):
                    cp.wait()
                scat(k, j).start(add=True)

                @pl.when(k >= 2)
                def _():
                    scat(k - 2, bn).wait()

                @pl.when(k <= nchunks - 4)
                def _():
                    for cp in load(k + 3, bn):
                        cp.start()

        for k in (nchunks - 2, nchunks - 1):
            scat(k, k % _NBUF).wait()

        plsc.subcore_barrier()
        pltpu.sync_copy(acc.at[pl.ds(zbase, _ZROWS)],
                        out_hbm.at[c, pl.ds(zbase, _ZROWS)])

    return k4


_K5_B = 1000


def _k5_body(*refs):
    acc_refs = refs[:_NCH]
    xup_ref, nf_ref, gn_ref, bn_ref, x_ref = refs[_NCH:]
    num = acc_refs[0][0]
    den = acc_refs[0][1]
    for a in acc_refs[1:]:
        num = num + a[0]
        den = den + a[1]
    h = num / (den + 1e-6)
    xx = xup_ref[...] + h
    mu = jnp.mean(xx, axis=1, keepdims=True)
    var = jnp.mean((xx - mu) ** 2, axis=1, keepdims=True)
    ln = (xx - mu) * lax.rsqrt(var + 1e-5) * gn_ref[...] + bn_ref[...]
    x_ref[...] = nf_ref[...] + ln * jax.nn.sigmoid(ln)


def _k5(accs, xup, node_feats, gn, bn):
    nb = N // _K5_B
    row = lambda i: (i, 0)
    fix = lambda i: (0, 0)
    acc_spec = pl.BlockSpec((2, _K5_B, D), lambda i: (0, i, 0))
    return pl.pallas_call(
        _k5_body,
        grid=(nb,),
        in_specs=(
            [acc_spec] * _NCH
            + [
                pl.BlockSpec((_K5_B, D), row),
                pl.BlockSpec((_K5_B, D), row),
                pl.BlockSpec((1, D), fix),
                pl.BlockSpec((1, D), fix),
            ]
        ),
        out_specs=pl.BlockSpec((_K5_B, D), row),
        out_shape=jax.ShapeDtypeStruct((N, D), F32),
    )(*accs, xup, node_feats, gn, bn)


_CHUNKS = (160000, 160000)
_COFFS = tuple(sum(_CHUNKS[:i]) for i in range(len(_CHUNKS)))
_NCH = len(_CHUNKS)
_K2CS = tuple(_make_k2(n, o) for n, o in zip(_CHUNKS, _COFFS))
_K4CS = tuple(_make_k4(n, o) for n, o in zip(_CHUNKS, _COFFS))


def kernel(node_feats, edge_feats, time_feats, edge_index,
           Wt, bt, Wsg, bsg, Wdg, bdg, Weg, beg,
           Wsu, bsu, Wdu, bdu, gn, bn, ge, be):
    row2 = lambda v: v.reshape(1, D)
    src = edge_index[0].astype(jnp.int32)
    dst = edge_index[1].astype(jnp.int32)

    tsrc, tdst, xup = _k1(node_feats, time_feats, Wt, row2(bt), Wsg, row2(bsg),
                          Wdg, row2(bdg), Wdu, row2(bdu), Wsu, row2(bsu))

    ybuf = None
    accs = []
    for ci in range(_NCH):
        gs, gd = _K2CS[ci](src, dst, tsrc, tdst)
        sn, ybuf = _k3(gs, gd, edge_feats, Weg,
                       row2(beg), row2(ge), row2(be), ybuf,
                       _COFFS[ci] // _K3_B)
        accs.append(_K4CS[ci](dst, sn))

    x = _k5(accs, xup, node_feats, row2(gn), row2(bn))
    return (x, ybuf)


# speedup vs baseline: 1.0605x; 1.0070x over previous
import functools

import jax
import jax.numpy as jnp
from jax import lax
from jax.experimental import pallas as pl
from jax.experimental.pallas import tpu as pltpu
from jax.experimental.pallas import tpu_sc as plsc

N = 10000
E = 320000
D = 128
NP = 10240

F32 = jnp.float32
BF16 = jnp.bfloat16
_DN = (((1,), (1,)), ((), ()))

_MESH = plsc.VectorSubcoreMesh(core_axis_name="c", subcore_axis_name="s")
NC, NS = 2, 16
NW = NC * NS


_K1_B = 2000


def _pack_pair(a, b):
    abits = lax.bitcast_convert_type(a.astype(BF16).astype(F32), jnp.uint32)
    bbits = lax.bitcast_convert_type(b.astype(BF16).astype(F32), jnp.uint32)
    lo = lax.shift_right_logical(abits, jnp.uint32(16))
    hi = jnp.bitwise_and(bbits, jnp.uint32(0xFFFF0000))
    return lax.bitcast_convert_type(jnp.bitwise_or(hi, lo), jnp.int32)


def _k1_body(x_ref, t_ref, wt_ref, bt_ref, wsg_ref, bsg_ref, wdg_ref, bdg_ref,
             wdu_ref, bdu_ref, wsu_ref, bsu_ref,
             tsrc_ref, tdst_ref, xup_ref):
    x = x_ref[...]
    tp = lax.dot_general(t_ref[...], wt_ref[...], _DN,
                         preferred_element_type=F32) + bt_ref[...]
    esrc = lax.dot_general(x, wsg_ref[...], _DN,
                           preferred_element_type=F32) + bsg_ref[...] + tp
    bh = lax.dot_general(x, wdu_ref[...], _DN,
                         preferred_element_type=F32) + bdu_ref[...]
    tsrc_ref[...] = _pack_pair(esrc, bh)
    tdst_ref[...] = lax.dot_general(x, wdg_ref[...], _DN,
                                    preferred_element_type=F32) + bdg_ref[...]
    xup_ref[...] = lax.dot_general(x, wsu_ref[...], _DN,
                                   preferred_element_type=F32) + bsu_ref[...]


def _k1(node_feats, time_feats, Wt, bt, Wsg, bsg, Wdg, bdg, Wdu, bdu, Wsu, bsu):
    nb = N // _K1_B
    row = lambda i: (i, 0)
    fix = lambda i: (0, 0)
    w_spec = pl.BlockSpec((D, D), fix)
    b_spec = pl.BlockSpec((1, D), fix)
    return pl.pallas_call(
        _k1_body,
        grid=(nb,),
        in_specs=[
            pl.BlockSpec((_K1_B, D), row), pl.BlockSpec((_K1_B, D), row),
            w_spec, b_spec, w_spec, b_spec, w_spec, b_spec,
            w_spec, b_spec, w_spec, b_spec,
        ],
        out_specs=[
            pl.BlockSpec((_K1_B, D), row),
            pl.BlockSpec((_K1_B, D), row),
            pl.BlockSpec((_K1_B, D), row),
        ],
        out_shape=[
            jax.ShapeDtypeStruct((N, D), jnp.int32),
            jax.ShapeDtypeStruct((N, D), F32),
            jax.ShapeDtypeStruct((N, D), F32),
        ],
    )(node_feats, time_feats, Wt, bt, Wsg, bsg, Wdg, bdg, Wdu, bdu, Wsu, bsu)


_K2_B = 40
_NBUF = 5


def _make_k2(ne, eoff):
    epw = ne // NW
    nchunks = epw // _K2_B

    @functools.partial(
        pl.kernel,
        out_type=(
            jax.ShapeDtypeStruct((ne, D), jnp.int32),
            jax.ShapeDtypeStruct((ne, D), F32),
        ),
        mesh=_MESH,
        scratch_types=(
            [pltpu.VMEM((epw,), jnp.int32)] * 2
            + [pltpu.VMEM((_K2_B, D), jnp.int32)] * _NBUF
            + [pltpu.VMEM((_K2_B, D), F32)] * _NBUF
            + [pltpu.SemaphoreType.DMA] * (4 * _NBUF)
        ),
    )
    def k2(src_hbm, dst_hbm, tsrc_hbm, tdst_hbm, gs_hbm, gd_hbm, *scr):
        isv, idv = scr[0], scr[1]
        rs = scr[2:2 + _NBUF]
        rd = scr[2 + _NBUF:2 + 2 * _NBUF]
        gssem = scr[2 + 2 * _NBUF:2 + 3 * _NBUF]
        gdsem = scr[2 + 3 * _NBUF:2 + 4 * _NBUF]
        wssem = scr[2 + 4 * _NBUF:2 + 5 * _NBUF]
        wdsem = scr[2 + 5 * _NBUF:2 + 6 * _NBUF]
        wid = lax.axis_index("s") * NC + lax.axis_index("c")
        ebase = wid * epw

        pltpu.sync_copy(src_hbm.at[pl.ds(eoff + ebase, epw)], isv)
        pltpu.sync_copy(dst_hbm.at[pl.ds(eoff + ebase, epw)], idv)

        def gather(c, b):
            sl = pl.ds(c * _K2_B, _K2_B)
            return (pltpu.make_async_copy(tsrc_hbm.at[isv.at[sl]], rs[b],
                                          gssem[b]),
                    pltpu.make_async_copy(tdst_hbm.at[idv.at[sl]], rd[b],
                                          gdsem[b]))

        def write(c, b):
            off = ebase + c * _K2_B
            return (pltpu.make_async_copy(rs[b], gs_hbm.at[pl.ds(off, _K2_B)],
                                          wssem[b]),
                    pltpu.make_async_copy(rd[b], gd_hbm.at[pl.ds(off, _K2_B)],
                                          wdsem[b]))

        for b in range(3):
            for cp in gather(b, b):
                cp.start()

        @pl.loop(0, nchunks // _NBUF)
        def _(t):
            for j in range(_NBUF):
                c = t * _NBUF + j
                bn = (j + 3) % _NBUF
                for cp in gather(c, j):
                    cp.wait()
                for cp in write(c, j):
                    cp.start()

                @pl.when(c >= 2)
                def _():
                    for cp in write(c - 2, bn):
                        cp.wait()

                @pl.when(c <= nchunks - 4)
                def _():
                    for cp in gather(c + 3, bn):
                        cp.start()

        for c in (nchunks - 2, nchunks - 1):
            for cp in write(c, c % _NBUF):
                cp.wait()

    return k2


_K3_B = 8000


def _k3_body(gs_ref, gd_ref, ef_ref, weg_ref, beg_ref, ge_ref,
             be_ref, sn_ref, y_ref):
    w = lax.bitcast_convert_type(gs_ref[...], jnp.uint32)
    esrc = lax.bitcast_convert_type(lax.shift_left(w, jnp.uint32(16)), F32)
    bh = lax.bitcast_convert_type(
        jnp.bitwise_and(w, jnp.uint32(0xFFFF0000)), F32)
    ef = ef_ref[...]
    eg = lax.dot_general(ef, weg_ref[...], _DN,
                         preferred_element_type=F32) + beg_ref[...]
    m = esrc + gd_ref[...] + eg
    sig = pl.reciprocal(1.0 + jnp.exp(-m), approx=True)
    sn_ref[0] = bh * sig
    sn_ref[1] = sig
    mu = jnp.mean(m, axis=1, keepdims=True)
    var = jnp.mean((m - mu) ** 2, axis=1, keepdims=True)
    ln = (m - mu) * lax.rsqrt(var + 1e-5) * ge_ref[...] + be_ref[...]
    y_ref[...] = ef + ln * pl.reciprocal(1.0 + jnp.exp(-ln), approx=True)


def _k3_body_alias(gs_ref, gd_ref, ef_ref, weg_ref, beg_ref, ge_ref,
                   be_ref, yin_ref, sn_ref, y_ref):
    _k3_body(gs_ref, gd_ref, ef_ref, weg_ref, beg_ref, ge_ref,
             be_ref, sn_ref, y_ref)


def _k3(gs, gd, edge_feats, Weg, beg, ge, be, ybuf, blk0):
    ne = gs.shape[0]
    nb = ne // _K3_B
    row = lambda i: (i + blk0, 0)
    crow = lambda i: (i, 0)
    fix = lambda i: (0, 0)
    in_specs = [
        pl.BlockSpec((_K3_B, D), crow),
        pl.BlockSpec((_K3_B, D), crow),
        pl.BlockSpec((_K3_B, D), row),
        pl.BlockSpec((D, D), fix),
        pl.BlockSpec((1, D), fix),
        pl.BlockSpec((1, D), fix),
        pl.BlockSpec((1, D), fix),
    ]
    args = [gs, gd, edge_feats, Weg, beg, ge, be]
    body = _k3_body
    aliases = {}
    if ybuf is not None:
        in_specs.append(pl.BlockSpec(memory_space=pl.ANY))
        args.append(ybuf)
        body = _k3_body_alias
        aliases = {7: 1}
    return pl.pallas_call(
        body,
        grid=(nb,),
        in_specs=in_specs,
        out_specs=[
            pl.BlockSpec((2, _K3_B, D), lambda i: (0, i, 0)),
            pl.BlockSpec((_K3_B, D), row),
        ],
        out_shape=[
            jax.ShapeDtypeStruct((2, ne, D), F32),
            jax.ShapeDtypeStruct((E, D), F32),
        ],
        input_output_aliases=aliases,
    )(*args)


_K4_B = 40
_ZROWS = NP // NS


def _make_k4(ne, eoff):
    eps = ne // NS
    nchunks = eps // _K4_B

    @functools.partial(
        pl.kernel,
        out_type=jax.ShapeDtypeStruct((NC, NP, D), F32),
        mesh=_MESH,
        scratch_types=(
            [pltpu.VMEM((_K4_B,), jnp.int32)] * _NBUF
            + [pltpu.VMEM((_K4_B, D), F32)] * _NBUF
            + [pltpu.VMEM_SHARED((NP, D), F32)]
            + [pltpu.SemaphoreType.DMA] * (3 * _NBUF)
        ),
    )
    def k4(dst_hbm, sn_hbm, out_hbm, *scr):
        idxv = scr[0:_NBUF]
        vals = scr[_NBUF:2 * _NBUF]
        acc = scr[2 * _NBUF]
        lsem = scr[1 + 2 * _NBUF:1 + 3 * _NBUF]
        isem = scr[1 + 3 * _NBUF:1 + 4 * _NBUF]
        ssem = scr[1 + 4 * _NBUF:1 + 5 * _NBUF]
        c = lax.axis_index("c")
        s = lax.axis_index("s")

        @pl.loop(0, _K4_B)
        def _(r):
            for j in range(D // 16):
                vals[0].at[r, pl.ds(j * 16, 16)][...] = jnp.zeros((16,), F32)

        zbase = s * _ZROWS
        @pl.loop(0, _ZROWS // _K4_B)
        def _(k):
            pltpu.sync_copy(vals[0], acc.at[pl.ds(zbase + k * _K4_B, _K4_B)])

        plsc.subcore_barrier()

        ebase = s * eps

        def load(k, b):
            off = ebase + k * _K4_B
            return (pltpu.make_async_copy(dst_hbm.at[pl.ds(eoff + off, _K4_B)],
                                          idxv[b], isem[b]),
                    pltpu.make_async_copy(sn_hbm.at[c, pl.ds(off, _K4_B)],
                                          vals[b], lsem[b]))

        def scat(k, b):
            return pltpu.make_async_copy(vals[b], acc.at[idxv[b]], ssem[b])

        for b in range(3):
            for cp in load(b, b):
                cp.start()

        @pl.loop(0, nchunks // _NBUF)
        def _(t):
            for j in range(_NBUF):
                k = t * _NBUF + j
                bn = (j + 3) % _NBUF
                for cp in load(k, j):
                    cp.wait()
                scat(k, j).start(add=True)

                @pl.when(k >= 2)
                def _():
                    scat(k - 2, bn).wait()

                @pl.when(k <= nchunks - 4)
                def _():
                    for cp in load(k + 3, bn):
                        cp.start()

        for k in (nchunks - 2, nchunks - 1):
            scat(k, k % _NBUF).wait()

        plsc.subcore_barrier()
        pltpu.sync_copy(acc.at[pl.ds(zbase, _ZROWS)],
                        out_hbm.at[c, pl.ds(zbase, _ZROWS)])

    return k4


_K5_B = 1000


def _k5_body(*refs):
    acc_refs = refs[:_NCH]
    xup_ref, nf_ref, gn_ref, bn_ref, x_ref = refs[_NCH:]
    num = acc_refs[0][0]
    den = acc_refs[0][1]
    for a in acc_refs[1:]:
        num = num + a[0]
        den = den + a[1]
    h = num / (den + 1e-6)
    xx = xup_ref[...] + h
    mu = jnp.mean(xx, axis=1, keepdims=True)
    var = jnp.mean((xx - mu) ** 2, axis=1, keepdims=True)
    ln = (xx - mu) * lax.rsqrt(var + 1e-5) * gn_ref[...] + bn_ref[...]
    x_ref[...] = nf_ref[...] + ln * jax.nn.sigmoid(ln)


def _k5(accs, xup, node_feats, gn, bn):
    nb = N // _K5_B
    row = lambda i: (i, 0)
    fix = lambda i: (0, 0)
    acc_spec = pl.BlockSpec((2, _K5_B, D), lambda i: (0, i, 0))
    return pl.pallas_call(
        _k5_body,
        grid=(nb,),
        in_specs=(
            [acc_spec] * _NCH
            + [
                pl.BlockSpec((_K5_B, D), row),
                pl.BlockSpec((_K5_B, D), row),
                pl.BlockSpec((1, D), fix),
                pl.BlockSpec((1, D), fix),
            ]
        ),
        out_specs=pl.BlockSpec((_K5_B, D), row),
        out_shape=jax.ShapeDtypeStruct((N, D), F32),
    )(*accs, xup, node_feats, gn, bn)


_CHUNKS = (160000, 160000)
_COFFS = tuple(sum(_CHUNKS[:i]) for i in range(len(_CHUNKS)))
_NCH = len(_CHUNKS)
_K2CS = tuple(_make_k2(n, o) for n, o in zip(_CHUNKS, _COFFS))
_K4CS = tuple(_make_k4(n, o) for n, o in zip(_CHUNKS, _COFFS))


def kernel(node_feats, edge_feats, time_feats, edge_index,
           Wt, bt, Wsg, bsg, Wdg, bdg, Weg, beg,
           Wsu, bsu, Wdu, bdu, gn, bn, ge, be):
    row2 = lambda v: v.reshape(1, D)
    src = edge_index[0].astype(jnp.int32)
    dst = edge_index[1].astype(jnp.int32)

    tsrc, tdst, xup = _k1(node_feats, time_feats, Wt, row2(bt), Wsg, row2(bsg),
                          Wdg, row2(bdg), Wdu, row2(bdu), Wsu, row2(bsu))

    ybuf = None
    accs = []
    for ci in range(_NCH):
        gs, gd = _K2CS[ci](src, dst, tsrc, tdst)
        sn, ybuf = _k3(gs, gd, edge_feats, Weg,
                       row2(beg), row2(ge), row2(be), ybuf,
                       _COFFS[ci] // _K3_B)
        accs.append(_K4CS[ci](dst, sn))

    x = _k5(accs, xup, node_feats, row2(gn), row2(bn))
    return (x, ybuf)


# speedup vs baseline: 1.0644x; 1.0036x over previous
import functools

import jax
import jax.numpy as jnp
from jax import lax
from jax.experimental import pallas as pl
from jax.experimental.pallas import tpu as pltpu
from jax.experimental.pallas import tpu_sc as plsc

N = 10000
E = 320000
D = 128
NP = 10240

F32 = jnp.float32
BF16 = jnp.bfloat16
_DN = (((1,), (1,)), ((), ()))

_MESH = plsc.VectorSubcoreMesh(core_axis_name="c", subcore_axis_name="s")
NC, NS = 2, 16
NW = NC * NS


_K1_B = 2000


def _pack_pair(a, b):
    abits = lax.bitcast_convert_type(a.astype(BF16).astype(F32), jnp.uint32)
    bbits = lax.bitcast_convert_type(b.astype(BF16).astype(F32), jnp.uint32)
    lo = lax.shift_right_logical(abits, jnp.uint32(16))
    hi = jnp.bitwise_and(bbits, jnp.uint32(0xFFFF0000))
    return lax.bitcast_convert_type(jnp.bitwise_or(hi, lo), jnp.int32)


def _k1_body(x_ref, t_ref, wt_ref, bt_ref, wsg_ref, bsg_ref, wdg_ref, bdg_ref,
             wdu_ref, bdu_ref, wsu_ref, bsu_ref,
             tsrc_ref, tdst_ref, xup_ref):
    x = x_ref[...]
    tp = lax.dot_general(t_ref[...], wt_ref[...], _DN,
                         preferred_element_type=F32) + bt_ref[...]
    esrc = lax.dot_general(x, wsg_ref[...], _DN,
                           preferred_element_type=F32) + bsg_ref[...] + tp
    bh = lax.dot_general(x, wdu_ref[...], _DN,
                         preferred_element_type=F32) + bdu_ref[...]
    tsrc_ref[...] = _pack_pair(esrc, bh)
    tdst_ref[...] = lax.dot_general(x, wdg_ref[...], _DN,
                                    preferred_element_type=F32) + bdg_ref[...]
    xup_ref[...] = lax.dot_general(x, wsu_ref[...], _DN,
                                   preferred_element_type=F32) + bsu_ref[...]


def _k1(node_feats, time_feats, Wt, bt, Wsg, bsg, Wdg, bdg, Wdu, bdu, Wsu, bsu):
    nb = N // _K1_B
    row = lambda i: (i, 0)
    fix = lambda i: (0, 0)
    w_spec = pl.BlockSpec((D, D), fix)
    b_spec = pl.BlockSpec((1, D), fix)
    return pl.pallas_call(
        _k1_body,
        grid=(nb,),
        in_specs=[
            pl.BlockSpec((_K1_B, D), row), pl.BlockSpec((_K1_B, D), row),
            w_spec, b_spec, w_spec, b_spec, w_spec, b_spec,
            w_spec, b_spec, w_spec, b_spec,
        ],
        out_specs=[
            pl.BlockSpec((_K1_B, D), row),
            pl.BlockSpec((_K1_B, D), row),
            pl.BlockSpec((_K1_B, D), row),
        ],
        out_shape=[
            jax.ShapeDtypeStruct((N, D), jnp.int32),
            jax.ShapeDtypeStruct((N, D), F32),
            jax.ShapeDtypeStruct((N, D), F32),
        ],
    )(node_feats, time_feats, Wt, bt, Wsg, bsg, Wdg, bdg, Wdu, bdu, Wsu, bsu)


_K2_B = 40
_NBUF = 5


def _make_k2(ne, eoff):
    epw = ne // NW
    nchunks = epw // _K2_B

    @functools.partial(
        pl.kernel,
        out_type=(
            jax.ShapeDtypeStruct((ne, D), jnp.int32),
            jax.ShapeDtypeStruct((ne, D), F32),
        ),
        mesh=_MESH,
        scratch_types=(
            [pltpu.VMEM((epw,), jnp.int32)] * 2
            + [pltpu.VMEM((_K2_B, D), jnp.int32)] * _NBUF
            + [pltpu.VMEM((_K2_B, D), F32)] * _NBUF
            + [pltpu.SemaphoreType.DMA] * (4 * _NBUF)
        ),
    )
    def k2(ei_hbm, tsrc_hbm, tdst_hbm, gs_hbm, gd_hbm, *scr):
        isv, idv = scr[0], scr[1]
        rs = scr[2:2 + _NBUF]
        rd = scr[2 + _NBUF:2 + 2 * _NBUF]
        gssem = scr[2 + 2 * _NBUF:2 + 3 * _NBUF]
        gdsem = scr[2 + 3 * _NBUF:2 + 4 * _NBUF]
        wssem = scr[2 + 4 * _NBUF:2 + 5 * _NBUF]
        wdsem = scr[2 + 5 * _NBUF:2 + 6 * _NBUF]
        wid = lax.axis_index("s") * NC + lax.axis_index("c")
        ebase = wid * epw

        pltpu.sync_copy(ei_hbm.at[pl.ds(eoff + ebase, epw)], isv)
        pltpu.sync_copy(ei_hbm.at[pl.ds(E + eoff + ebase, epw)], idv)

        def gather(c, b):
            sl = pl.ds(c * _K2_B, _K2_B)
            return (pltpu.make_async_copy(tsrc_hbm.at[isv.at[sl]], rs[b],
                                          gssem[b]),
                    pltpu.make_async_copy(tdst_hbm.at[idv.at[sl]], rd[b],
                                          gdsem[b]))

        def write(c, b):
            off = ebase + c * _K2_B
            return (pltpu.make_async_copy(rs[b], gs_hbm.at[pl.ds(off, _K2_B)],
                                          wssem[b]),
                    pltpu.make_async_copy(rd[b], gd_hbm.at[pl.ds(off, _K2_B)],
                                          wdsem[b]))

        for b in range(3):
            for cp in gather(b, b):
                cp.start()

        @pl.loop(0, nchunks // _NBUF)
        def _(t):
            for j in range(_NBUF):
                c = t * _NBUF + j
                bn = (j + 3) % _NBUF
                for cp in gather(c, j):
                    cp.wait()
                for cp in write(c, j):
                    cp.start()

                @pl.when(c >= 2)
                def _():
                    for cp in write(c - 2, bn):
                        cp.wait()

                @pl.when(c <= nchunks - 4)
                def _():
                    for cp in gather(c + 3, bn):
                        cp.start()

        for c in (nchunks - 2, nchunks - 1):
            for cp in write(c, c % _NBUF):
                cp.wait()

    return k2


_K3_B = 8000


def _k3_body(gs_ref, gd_ref, ef_ref, weg_ref, beg_ref, ge_ref,
             be_ref, sn_ref, y_ref):
    w = lax.bitcast_convert_type(gs_ref[...], jnp.uint32)
    esrc = lax.bitcast_convert_type(lax.shift_left(w, jnp.uint32(16)), F32)
    bh = lax.bitcast_convert_type(
        jnp.bitwise_and(w, jnp.uint32(0xFFFF0000)), F32)
    ef = ef_ref[...]
    eg = lax.dot_general(ef, weg_ref[...], _DN,
                         preferred_element_type=F32) + beg_ref[...]
    m = esrc + gd_ref[...] + eg
    sig = pl.reciprocal(1.0 + jnp.exp(-m), approx=True)
    sn_ref[0] = bh * sig
    sn_ref[1] = sig
    mu = jnp.mean(m, axis=1, keepdims=True)
    var = jnp.mean((m - mu) ** 2, axis=1, keepdims=True)
    ln = (m - mu) * lax.rsqrt(var + 1e-5) * ge_ref[...] + be_ref[...]
    y_ref[...] = ef + ln * pl.reciprocal(1.0 + jnp.exp(-ln), approx=True)


def _k3_body_alias(gs_ref, gd_ref, ef_ref, weg_ref, beg_ref, ge_ref,
                   be_ref, yin_ref, sn_ref, y_ref):
    _k3_body(gs_ref, gd_ref, ef_ref, weg_ref, beg_ref, ge_ref,
             be_ref, sn_ref, y_ref)


def _k3(gs, gd, edge_feats, Weg, beg, ge, be, ybuf, blk0, lblk0, nsub):
    ne = nsub
    nb = ne // _K3_B
    row = lambda i: (i + blk0, 0)
    crow = lambda i: (i + lblk0, 0)
    fix = lambda i: (0, 0)
    in_specs = [
        pl.BlockSpec((_K3_B, D), crow),
        pl.BlockSpec((_K3_B, D), crow),
        pl.BlockSpec((_K3_B, D), row),
        pl.BlockSpec((D, D), fix),
        pl.BlockSpec((1, D), fix),
        pl.BlockSpec((1, D), fix),
        pl.BlockSpec((1, D), fix),
    ]
    args = [gs, gd, edge_feats, Weg, beg, ge, be]
    body = _k3_body
    aliases = {}
    if ybuf is not None:
        in_specs.append(pl.BlockSpec(memory_space=pl.ANY))
        args.append(ybuf)
        body = _k3_body_alias
        aliases = {7: 1}
    return pl.pallas_call(
        body,
        grid=(nb,),
        in_specs=in_specs,
        out_specs=[
            pl.BlockSpec((2, _K3_B, D), lambda i: (0, i, 0)),
            pl.BlockSpec((_K3_B, D), row),
        ],
        out_shape=[
            jax.ShapeDtypeStruct((2, ne, D), F32),
            jax.ShapeDtypeStruct((E, D), F32),
        ],
        input_output_aliases=aliases,
    )(*args)


_K4_B = 40
_ZROWS = NP // NS


def _make_k4(ne, eoff):
    eps = ne // NS
    nchunks = eps // _K4_B

    @functools.partial(
        pl.kernel,
        out_type=jax.ShapeDtypeStruct((NC, NP, D), F32),
        mesh=_MESH,
        scratch_types=(
            [pltpu.VMEM((_K4_B,), jnp.int32)] * _NBUF
            + [pltpu.VMEM((_K4_B, D), F32)] * _NBUF
            + [pltpu.VMEM_SHARED((NP, D), F32)]
            + [pltpu.SemaphoreType.DMA] * (3 * _NBUF)
        ),
    )
    def k4(ei_hbm, sn_hbm, out_hbm, *scr):
        idxv = scr[0:_NBUF]
        vals = scr[_NBUF:2 * _NBUF]
        acc = scr[2 * _NBUF]
        lsem = scr[1 + 2 * _NBUF:1 + 3 * _NBUF]
        isem = scr[1 + 3 * _NBUF:1 + 4 * _NBUF]
        ssem = scr[1 + 4 * _NBUF:1 + 5 * _NBUF]
        c = lax.axis_index("c")
        s = lax.axis_index("s")

        @pl.loop(0, _K4_B)
        def _(r):
            for j in range(D // 16):
                vals[0].at[r, pl.ds(j * 16, 16)][...] = jnp.zeros((16,), F32)

        zbase = s * _ZROWS
        @pl.loop(0, _ZROWS // _K4_B)
        def _(k):
            pltpu.sync_copy(vals[0], acc.at[pl.ds(zbase + k * _K4_B, _K4_B)])

        plsc.subcore_barrier()

        ebase = s * eps

        def load(k, b):
            off = ebase + k * _K4_B
            return (pltpu.make_async_copy(ei_hbm.at[pl.ds(E + eoff + off, _K4_B)],
                                          idxv[b], isem[b]),
                    pltpu.make_async_copy(sn_hbm.at[c, pl.ds(off, _K4_B)],
                                          vals[b], lsem[b]))

        def scat(k, b):
            return pltpu.make_async_copy(vals[b], acc.at[idxv[b]], ssem[b])

        for b in range(3):
            for cp in load(b, b):
                cp.start()

        @pl.loop(0, nchunks // _NBUF)
        def _(t):
            for j in range(_NBUF):
                k = t * _NBUF + j
                bn = (j + 3) % _NBUF
                for cp in load(k, j):
                    cp.wait()
                scat(k, j).start(add=True)

                @pl.when(k >= 2)
                def _():
                    scat(k - 2, bn).wait()

                @pl.when(k <= nchunks - 4)
                def _():
                    for cp in load(k + 3, bn):
                        cp.start()

        for k in (nchunks - 2, nchunks - 1):
            scat(k, k % _NBUF).wait()

        plsc.subcore_barrier()
        pltpu.sync_copy(acc.at[pl.ds(zbase, _ZROWS)],
                        out_hbm.at[c, pl.ds(zbase, _ZROWS)])

    return k4


_K5_B = 1000


def _k5_body(*refs):
    acc_refs = refs[:_NCH]
    xup_ref, nf_ref, gn_ref, bn_ref, x_ref = refs[_NCH:]
    num = acc_refs[0][0]
    den = acc_refs[0][1]
    for a in acc_refs[1:]:
        num = num + a[0]
        den = den + a[1]
    h = num / (den + 1e-6)
    xx = xup_ref[...] + h
    mu = jnp.mean(xx, axis=1, keepdims=True)
    var = jnp.mean((xx - mu) ** 2, axis=1, keepdims=True)
    ln = (xx - mu) * lax.rsqrt(var + 1e-5) * gn_ref[...] + bn_ref[...]
    x_ref[...] = nf_ref[...] + ln * jax.nn.sigmoid(ln)


def _k5(accs, xup, node_feats, gn, bn):
    nb = N // _K5_B
    row = lambda i: (i, 0)
    fix = lambda i: (0, 0)
    acc_spec = pl.BlockSpec((2, _K5_B, D), lambda i: (0, i, 0))
    return pl.pallas_call(
        _k5_body,
        grid=(nb,),
        in_specs=(
            [acc_spec] * _NCH
            + [
                pl.BlockSpec((_K5_B, D), row),
                pl.BlockSpec((_K5_B, D), row),
                pl.BlockSpec((1, D), fix),
                pl.BlockSpec((1, D), fix),
            ]
        ),
        out_specs=pl.BlockSpec((_K5_B, D), row),
        out_shape=jax.ShapeDtypeStruct((N, D), F32),
    )(*accs, xup, node_feats, gn, bn)


_PLAN = (
    (160000, 0, ((0, 160000),)),
    (160000, 160000, ((160000, 80000), (240000, 80000))),
)
_NCH = sum(len(p[2]) for p in _PLAN)
_K2CS = tuple(_make_k2(n, o) for n, o, _ in _PLAN)
_K4CS = {(o, n): _make_k4(n, o)
         for _, _, subs in _PLAN for o, n in subs}


def kernel(node_feats, edge_feats, time_feats, edge_index,
           Wt, bt, Wsg, bsg, Wdg, bdg, Weg, beg,
           Wsu, bsu, Wdu, bdu, gn, bn, ge, be):
    row2 = lambda v: v.reshape(1, D)
    ei = edge_index.reshape(2 * E)

    tsrc, tdst, xup = _k1(node_feats, time_feats, Wt, row2(bt), Wsg, row2(bsg),
                          Wdg, row2(bdg), Wdu, row2(bdu), Wsu, row2(bsu))

    ybuf = None
    accs = []
    for ci, (n2, o2, subs) in enumerate(_PLAN):
        gs, gd = _K2CS[ci](ei, tsrc, tdst)
        for o3, n3 in subs:
            sn, ybuf = _k3(gs, gd, edge_feats, Weg,
                           row2(beg), row2(ge), row2(be), ybuf,
                           o3 // _K3_B, (o3 - o2) // _K3_B, n3)
            accs.append(_K4CS[(o3, n3)](ei, sn))

    x = _k5(accs, xup, node_feats, row2(gn), row2(bn))
    return (x, ybuf)


# speedup vs baseline: 1.0783x; 1.0131x over previous
import functools

import jax
import jax.numpy as jnp
from jax import lax
from jax.experimental import pallas as pl
from jax.experimental.pallas import tpu as pltpu
from jax.experimental.pallas import tpu_sc as plsc

N = 10000
E = 320000
D = 128
NP = 10240

F32 = jnp.float32
BF16 = jnp.bfloat16
_DN = (((1,), (1,)), ((), ()))

_MESH = plsc.VectorSubcoreMesh(core_axis_name="c", subcore_axis_name="s")
NC, NS = 2, 16
NW = NC * NS


_K1_B = 2000


def _pack_pair(a, b):
    abits = lax.bitcast_convert_type(a.astype(BF16).astype(F32), jnp.uint32)
    bbits = lax.bitcast_convert_type(b.astype(BF16).astype(F32), jnp.uint32)
    lo = lax.shift_right_logical(abits, jnp.uint32(16))
    hi = jnp.bitwise_and(bbits, jnp.uint32(0xFFFF0000))
    return lax.bitcast_convert_type(jnp.bitwise_or(hi, lo), jnp.int32)


def _k1_body(x_ref, t_ref, wt_ref, bt_ref, wsg_ref, bsg_ref, wdg_ref, bdg_ref,
             wdu_ref, bdu_ref, wsu_ref, bsu_ref,
             tsrc_ref, tdst_ref, xup_ref):
    x = x_ref[...]
    tp = lax.dot_general(t_ref[...], wt_ref[...], _DN,
                         preferred_element_type=F32) + bt_ref[...]
    esrc = lax.dot_general(x, wsg_ref[...], _DN,
                           preferred_element_type=F32) + bsg_ref[...] + tp
    bh = lax.dot_general(x, wdu_ref[...], _DN,
                         preferred_element_type=F32) + bdu_ref[...]
    tsrc_ref[...] = _pack_pair(esrc, bh)
    tdst_ref[...] = lax.dot_general(x, wdg_ref[...], _DN,
                                    preferred_element_type=F32) + bdg_ref[...]
    xup_ref[...] = lax.dot_general(x, wsu_ref[...], _DN,
                                   preferred_element_type=F32) + bsu_ref[...]


def _k1(node_feats, time_feats, Wt, bt, Wsg, bsg, Wdg, bdg, Wdu, bdu, Wsu, bsu):
    nb = N // _K1_B
    row = lambda i: (i, 0)
    fix = lambda i: (0, 0)
    w_spec = pl.BlockSpec((D, D), fix)
    b_spec = pl.BlockSpec((1, D), fix)
    return pl.pallas_call(
        _k1_body,
        grid=(nb,),
        in_specs=[
            pl.BlockSpec((_K1_B, D), row), pl.BlockSpec((_K1_B, D), row),
            w_spec, b_spec, w_spec, b_spec, w_spec, b_spec,
            w_spec, b_spec, w_spec, b_spec,
        ],
        out_specs=[
            pl.BlockSpec((_K1_B, D), row),
            pl.BlockSpec((_K1_B, D), row),
            pl.BlockSpec((_K1_B, D), row),
        ],
        out_shape=[
            jax.ShapeDtypeStruct((N, D), jnp.int32),
            jax.ShapeDtypeStruct((N, D), F32),
            jax.ShapeDtypeStruct((N, D), F32),
        ],
    )(node_feats, time_feats, Wt, bt, Wsg, bsg, Wdg, bdg, Wdu, bdu, Wsu, bsu)


_K2_B = 40
_NBUF = 5


def _make_k2(ne, eoff):
    epw = ne // NW
    nchunks = epw // _K2_B

    @functools.partial(
        pl.kernel,
        out_type=(
            jax.ShapeDtypeStruct((ne, D), jnp.int32),
            jax.ShapeDtypeStruct((ne, D), F32),
        ),
        mesh=_MESH,
        scratch_types=(
            [pltpu.VMEM((epw,), jnp.int32)] * 2
            + [pltpu.VMEM((_K2_B, D), jnp.int32)] * _NBUF
            + [pltpu.VMEM((_K2_B, D), F32)] * _NBUF
            + [pltpu.SemaphoreType.DMA] * (4 * _NBUF)
        ),
    )
    def k2(ei_hbm, tsrc_hbm, tdst_hbm, gs_hbm, gd_hbm, *scr):
        isv, idv = scr[0], scr[1]
        rs = scr[2:2 + _NBUF]
        rd = scr[2 + _NBUF:2 + 2 * _NBUF]
        gssem = scr[2 + 2 * _NBUF:2 + 3 * _NBUF]
        gdsem = scr[2 + 3 * _NBUF:2 + 4 * _NBUF]
        wssem = scr[2 + 4 * _NBUF:2 + 5 * _NBUF]
        wdsem = scr[2 + 5 * _NBUF:2 + 6 * _NBUF]
        wid = lax.axis_index("s") * NC + lax.axis_index("c")
        ebase = wid * epw

        pltpu.sync_copy(ei_hbm.at[pl.ds(eoff + ebase, epw)], isv)
        pltpu.sync_copy(ei_hbm.at[pl.ds(E + eoff + ebase, epw)], idv)

        def gather(c, b):
            sl = pl.ds(c * _K2_B, _K2_B)
            return (pltpu.make_async_copy(tsrc_hbm.at[isv.at[sl]], rs[b],
                                          gssem[b]),
                    pltpu.make_async_copy(tdst_hbm.at[idv.at[sl]], rd[b],
                                          gdsem[b]))

        def write(c, b):
            off = ebase + c * _K2_B
            return (pltpu.make_async_copy(rs[b], gs_hbm.at[pl.ds(off, _K2_B)],
                                          wssem[b]),
                    pltpu.make_async_copy(rd[b], gd_hbm.at[pl.ds(off, _K2_B)],
                                          wdsem[b]))

        for b in range(3):
            for cp in gather(b, b):
                cp.start()

        @pl.loop(0, nchunks // _NBUF)
        def _(t):
            for j in range(_NBUF):
                c = t * _NBUF + j
                bn = (j + 3) % _NBUF
                for cp in gather(c, j):
                    cp.wait()
                for cp in write(c, j):
                    cp.start()

                @pl.when(c >= 2)
                def _():
                    for cp in write(c - 2, bn):
                        cp.wait()

                @pl.when(c <= nchunks - 4)
                def _():
                    for cp in gather(c + 3, bn):
                        cp.start()

        for c in (nchunks - 2, nchunks - 1):
            for cp in write(c, c % _NBUF):
                cp.wait()

    return k2


_K3_B = 8000


def _k3_body(gs_ref, gd_ref, ef_ref, weg_ref, beg_ref, ge_ref,
             be_ref, sn_ref, y_ref):
    w = lax.bitcast_convert_type(gs_ref[...], jnp.uint32)
    esrc = lax.bitcast_convert_type(lax.shift_left(w, jnp.uint32(16)), F32)
    bh = lax.bitcast_convert_type(
        jnp.bitwise_and(w, jnp.uint32(0xFFFF0000)), F32)
    ef = ef_ref[...]
    eg = lax.dot_general(ef, weg_ref[...], _DN,
                         preferred_element_type=F32) + beg_ref[...]
    m = esrc + gd_ref[...] + eg
    sig = pl.reciprocal(1.0 + jnp.exp(-m), approx=True)
    sn_ref[0] = bh * sig
    sn_ref[1] = sig
    mu = jnp.mean(m, axis=1, keepdims=True)
    var = jnp.mean((m - mu) ** 2, axis=1, keepdims=True)
    ln = (m - mu) * lax.rsqrt(var + 1e-5) * ge_ref[...] + be_ref[...]
    y_ref[...] = ef + ln * pl.reciprocal(1.0 + jnp.exp(-ln), approx=True)


def _k3_body_alias(gs_ref, gd_ref, ef_ref, weg_ref, beg_ref, ge_ref,
                   be_ref, yin_ref, sn_ref, y_ref):
    _k3_body(gs_ref, gd_ref, ef_ref, weg_ref, beg_ref, ge_ref,
             be_ref, sn_ref, y_ref)


def _k3(gs, gd, edge_feats, Weg, beg, ge, be, ybuf, blk0, lblk0, nsub):
    ne = nsub
    nb = ne // _K3_B
    row = lambda i: (i + blk0, 0)
    crow = lambda i: (i + lblk0, 0)
    fix = lambda i: (0, 0)
    in_specs = [
        pl.BlockSpec((_K3_B, D), crow),
        pl.BlockSpec((_K3_B, D), crow),
        pl.BlockSpec((_K3_B, D), row),
        pl.BlockSpec((D, D), fix),
        pl.BlockSpec((1, D), fix),
        pl.BlockSpec((1, D), fix),
        pl.BlockSpec((1, D), fix),
    ]
    args = [gs, gd, edge_feats, Weg, beg, ge, be]
    body = _k3_body
    aliases = {}
    if ybuf is not None:
        in_specs.append(pl.BlockSpec(memory_space=pl.ANY))
        args.append(ybuf)
        body = _k3_body_alias
        aliases = {7: 1}
    return pl.pallas_call(
        body,
        grid=(nb,),
        in_specs=in_specs,
        out_specs=[
            pl.BlockSpec((2, _K3_B, D), lambda i: (0, i, 0)),
            pl.BlockSpec((_K3_B, D), row),
        ],
        out_shape=[
            jax.ShapeDtypeStruct((2, ne, D), F32),
            jax.ShapeDtypeStruct((E, D), F32),
        ],
        input_output_aliases=aliases,
    )(*args)


_K4_B = 40
_ZROWS = NP // NS


def _make_k4(ne, eoff):
    eps = ne // NS
    nchunks = eps // _K4_B

    @functools.partial(
        pl.kernel,
        out_type=jax.ShapeDtypeStruct((NC, NP, D), F32),
        mesh=_MESH,
        scratch_types=(
            [pltpu.VMEM((_K4_B,), jnp.int32)] * _NBUF
            + [pltpu.VMEM((_K4_B, D), F32)] * _NBUF
            + [pltpu.VMEM_SHARED((NP, D), F32)]
            + [pltpu.SemaphoreType.DMA] * (3 * _NBUF)
        ),
    )
    def k4(ei_hbm, sn_hbm, out_hbm, *scr):
        idxv = scr[0:_NBUF]
        vals = scr[_NBUF:2 * _NBUF]
        acc = scr[2 * _NBUF]
        lsem = scr[1 + 2 * _NBUF:1 + 3 * _NBUF]
        isem = scr[1 + 3 * _NBUF:1 + 4 * _NBUF]
        ssem = scr[1 + 4 * _NBUF:1 + 5 * _NBUF]
        c = lax.axis_index("c")
        s = lax.axis_index("s")

        @pl.loop(0, _K4_B)
        def _(r):
            for j in range(D // 16):
                vals[0].at[r, pl.ds(j * 16, 16)][...] = jnp.zeros((16,), F32)

        zbase = s * _ZROWS
        @pl.loop(0, _ZROWS // _K4_B)
        def _(k):
            pltpu.sync_copy(vals[0], acc.at[pl.ds(zbase + k * _K4_B, _K4_B)])

        plsc.subcore_barrier()

        ebase = s * eps

        def load(k, b):
            off = ebase + k * _K4_B
            return (pltpu.make_async_copy(ei_hbm.at[pl.ds(E + eoff + off, _K4_B)],
                                          idxv[b], isem[b]),
                    pltpu.make_async_copy(sn_hbm.at[c, pl.ds(off, _K4_B)],
                                          vals[b], lsem[b]))

        def scat(k, b):
            return pltpu.make_async_copy(vals[b], acc.at[idxv[b]], ssem[b])

        for b in range(3):
            for cp in load(b, b):
                cp.start()

        @pl.loop(0, nchunks // _NBUF)
        def _(t):
            for j in range(_NBUF):
                k = t * _NBUF + j
                bn = (j + 3) % _NBUF
                for cp in load(k, j):
                    cp.wait()
                scat(k, j).start(add=True)

                @pl.when(k >= 2)
                def _():
                    scat(k - 2, bn).wait()

                @pl.when(k <= nchunks - 4)
                def _():
                    for cp in load(k + 3, bn):
                        cp.start()

        for k in (nchunks - 2, nchunks - 1):
            scat(k, k % _NBUF).wait()

        plsc.subcore_barrier()
        pltpu.sync_copy(acc.at[pl.ds(zbase, _ZROWS)],
                        out_hbm.at[c, pl.ds(zbase, _ZROWS)])

    return k4


_K5_B = 1000


def _k5_body(*refs):
    acc_refs = refs[:_NCH]
    xup_ref, nf_ref, gn_ref, bn_ref, x_ref = refs[_NCH:]
    num = acc_refs[0][0]
    den = acc_refs[0][1]
    for a in acc_refs[1:]:
        num = num + a[0]
        den = den + a[1]
    h = num / (den + 1e-6)
    xx = xup_ref[...] + h
    mu = jnp.mean(xx, axis=1, keepdims=True)
    var = jnp.mean((xx - mu) ** 2, axis=1, keepdims=True)
    ln = (xx - mu) * lax.rsqrt(var + 1e-5) * gn_ref[...] + bn_ref[...]
    x_ref[...] = nf_ref[...] + ln * jax.nn.sigmoid(ln)


def _k5(accs, xup, node_feats, gn, bn):
    nb = N // _K5_B
    row = lambda i: (i, 0)
    fix = lambda i: (0, 0)
    acc_spec = pl.BlockSpec((2, _K5_B, D), lambda i: (0, i, 0))
    return pl.pallas_call(
        _k5_body,
        grid=(nb,),
        in_specs=(
            [acc_spec] * _NCH
            + [
                pl.BlockSpec((_K5_B, D), row),
                pl.BlockSpec((_K5_B, D), row),
                pl.BlockSpec((1, D), fix),
                pl.BlockSpec((1, D), fix),
            ]
        ),
        out_specs=pl.BlockSpec((_K5_B, D), row),
        out_shape=jax.ShapeDtypeStruct((N, D), F32),
    )(*accs, xup, node_feats, gn, bn)


_PLAN = (
    (160000, 0, ((0, 160000),)),
    (160000, 160000, ((160000, 160000),)),
)
_NCH = sum(len(p[2]) for p in _PLAN)
_K2CS = tuple(_make_k2(n, o) for n, o, _ in _PLAN)
_K4CS = {(o, n): _make_k4(n, o)
         for _, _, subs in _PLAN for o, n in subs}


def kernel(node_feats, edge_feats, time_feats, edge_index,
           Wt, bt, Wsg, bsg, Wdg, bdg, Weg, beg,
           Wsu, bsu, Wdu, bdu, gn, bn, ge, be):
    row2 = lambda v: v.reshape(1, D)
    ei = edge_index.reshape(2 * E)

    tsrc, tdst, xup = _k1(node_feats, time_feats, Wt, row2(bt), Wsg, row2(bsg),
                          Wdg, row2(bdg), Wdu, row2(bdu), Wsu, row2(bsu))

    ybuf = None
    accs = []
    for ci, (n2, o2, subs) in enumerate(_PLAN):
        gs, gd = _K2CS[ci](ei, tsrc, tdst)
        for o3, n3 in subs:
            sn, ybuf = _k3(gs, gd, edge_feats, Weg,
                           row2(beg), row2(ge), row2(be), ybuf,
                           o3 // _K3_B, (o3 - o2) // _K3_B, n3)
            accs.append(_K4CS[(o3, n3)](ei, sn))

    x = _k5(accs, xup, node_feats, row2(gn), row2(bn))
    return (x, ybuf)
